# baseline (speedup 1.0000x reference)
import jax
import jax.numpy as jnp
from jax import lax
from jax.experimental import pallas as pl
from jax.experimental.pallas import tpu as pltpu
from jax.experimental.pallas import tpu_sc as plsc

PAIR_DIM = 32
B = 32
N = 255
NP1 = 256
NUM_CORES = 2
NUM_SUBCORES = 16
L = 16

KB = 3
NIT = N // KB
SPTOT = B * N * N
EDTOT = B * N * N * 3
SPW = 784
EDW = 2320


def _sc_body(spf, edf, w_edge, w_sp, w_vnode, out,
             tsp, ted, vno, spw, edw, blk, ssem, esem, osem):
    b = lax.axis_index("s") * NUM_CORES + lax.axis_index("c")

    pltpu.sync_copy(w_sp, tsp)
    pltpu.sync_copy(w_edge, ted)
    pltpu.sync_copy(w_vnode, vno)

    v0 = vno[0, pl.ds(0, L)]
    v1 = vno[0, pl.ds(L, L)]

    def fill(j, _):
        blk[0, 0, j, pl.ds(0, L)] = v0
        blk[0, 0, j, pl.ds(L, L)] = v1
        return 0

    lax.fori_loop(0, NP1, fill, 0)
    pltpu.sync_copy(blk.at[0, 0], out.at[b, 0])

    third = jnp.float32(1.0 / 3.0)
    iota = lax.iota(jnp.int32, L)
    iota3 = iota * 3

    def sp_window(t):
        s0 = (b * N + t * KB) * N
        a0 = jnp.minimum(s0 - lax.rem(s0, 16), SPTOT - SPW)
        return pl.multiple_of(a0, 16), s0

    def ed_window(t):
        e0 = (b * N + t * KB) * (3 * N)
        b0 = jnp.minimum(e0 - lax.rem(e0, 16), EDTOT - EDW)
        return pl.multiple_of(b0, 16), e0

    def issue_idx(t, s):
        a0, _ = sp_window(t)
        b0, _ = ed_window(t)
        pltpu.async_copy(spf.at[pl.ds(a0, SPW)], spw.at[s], ssem.at[s])
        pltpu.async_copy(edf.at[pl.ds(b0, EDW)], edw.at[s], esem.at[s])

    def wait_idx(s):
        pltpu.make_async_copy(spf.at[pl.ds(0, SPW)], spw.at[s], ssem.at[s]).wait()
        pltpu.make_async_copy(edf.at[pl.ds(0, EDW)], edw.at[s], esem.at[s]).wait()

    def wait_out(s):
        pltpu.make_async_copy(blk.at[s], out.at[b, pl.ds(1, KB)], osem.at[s]).wait()

    issue_idx(0, 0)

    def step(t, _):
        slot = lax.rem(t, 2)
        nxt = 1 - slot

        @pl.when(t + 1 < NIT)
        def _():
            issue_idx(t + 1, nxt)

        a0, s0 = sp_window(t)
        b0, e0 = ed_window(t)
        sh_sp = s0 - a0
        sh_ed = e0 - b0

        wait_idx(slot)

        @pl.when(t >= 2)
        def _():
            wait_out(slot)

        for kk in range(KB):

            def group(g, _, kk=kk):
                j0 = g * L
                p = sh_sp + kk * N + (j0 - 1)
                spr = plsc.load_gather(spw.at[slot], [jnp.maximum(p + iota, 0)])
                ev = jnp.maximum(sh_ed + 3 * (kk * N + j0 - 1) + iota3, 0)
                i0 = plsc.load_gather(edw.at[slot], [ev])
                i1 = plsc.load_gather(edw.at[slot], [ev + 1])
                i2 = plsc.load_gather(edw.at[slot], [ev + 2])
                for l in range(L):
                    j = j0 + l
                    spv = spr[l]
                    e0v = i0[l]
                    e1v = i1[l]
                    e2v = i2[l]
                    acc0 = tsp[spv, pl.ds(0, L)] + third * (
                        ted[e0v, pl.ds(0, L)]
                        + ted[e1v, pl.ds(0, L)]
                        + ted[e2v, pl.ds(0, L)]
                    )
                    acc1 = tsp[spv, pl.ds(L, L)] + third * (
                        ted[e0v, pl.ds(L, L)]
                        + ted[e1v, pl.ds(L, L)]
                        + ted[e2v, pl.ds(L, L)]
                    )
                    blk[slot, kk, j, pl.ds(0, L)] = acc0
                    blk[slot, kk, j, pl.ds(L, L)] = acc1
                return 0

            lax.fori_loop(0, NP1 // L, group, 0)
            blk[slot, kk, 0, pl.ds(0, L)] = v0
            blk[slot, kk, 0, pl.ds(L, L)] = v1

        pltpu.async_copy(
            blk.at[slot], out.at[b, pl.ds(t * KB + 1, KB)], osem.at[slot]
        )
        return 0

    lax.fori_loop(0, NIT, step, 0)
    wait_out(1)
    wait_out(0)


@jax.jit
def kernel(shortest_path, edge_feat, graph_attn_bias, W_edge, W_sp, W_vnode):
    del graph_attn_bias
    spf = shortest_path.reshape(SPTOT)
    edf = edge_feat.reshape(EDTOT)

    mesh = plsc.VectorSubcoreMesh(
        core_axis_name="c", subcore_axis_name="s",
        num_cores=NUM_CORES, num_subcores=NUM_SUBCORES,
    )
    run = pl.kernel(
        _sc_body,
        out_type=jax.ShapeDtypeStruct((B, NP1, NP1, PAIR_DIM), jnp.float32),
        mesh=mesh,
        compiler_params=pltpu.CompilerParams(
            needs_layout_passes=False,
            use_tc_tiling_on_sc=False,
            disable_bounds_checks=True,
        ),
        scratch_types=[
            pltpu.VMEM((512, PAIR_DIM), jnp.float32),
            pltpu.VMEM((1024, PAIR_DIM), jnp.float32),
            pltpu.VMEM((1, PAIR_DIM), jnp.float32),
            pltpu.VMEM((2, SPW), jnp.int32),
            pltpu.VMEM((2, EDW), jnp.int32),
            pltpu.VMEM((2, KB, NP1, PAIR_DIM), jnp.float32),
            pltpu.SemaphoreType.DMA((2,)),
            pltpu.SemaphoreType.DMA((2,)),
            pltpu.SemaphoreType.DMA((2,)),
        ],
    )
    return run(spf, edf, W_edge, W_sp, W_vnode)
